# baseline (device time: 165085 ns/iter reference)
import functools

import jax
import jax.numpy as jnp
from jax import lax
from jax.experimental import pallas as pl
from jax.experimental.pallas import tpu as pltpu

N_DEV = 16
N_TOK = 1024
D_IN = 512
D_OUT = 1024
E_LOCAL = 4
CHUNK = N_TOK // N_DEV


def kernel(x, router_W, route_idx, expert_W, shared_W):
    def body(x_ref, rw_ref, idx_ref, ew_ref, sw_ref, out_ref,
             rs_buf, rs_send, rs_recv, ag_send, ag_recv, mid_sem):
        p = lax.axis_index("i")
        left = lax.rem(p + N_DEV - 1, N_DEV)
        right = lax.rem(p + 1, N_DEV)

        barrier_sem = pltpu.get_barrier_semaphore()
        for nbr in (left, right):
            pl.semaphore_signal(
                barrier_sem, inc=1,
                device_id=(nbr,), device_id_type=pl.DeviceIdType.MESH,
            )
        pl.semaphore_wait(barrier_sem, 2)

        xv = x_ref[:, :]
        scores = jnp.dot(xv, rw_ref[:, :], preferred_element_type=jnp.float32)
        m = jnp.max(scores, axis=1, keepdims=True)
        gate = 1.0 / jnp.sum(jnp.exp(scores - m), axis=1, keepdims=True)
        idx = idx_ref[:, :]
        base = p * E_LOCAL
        acc = None
        for k in range(E_LOCAL):
            sel = (idx == base + k).astype(jnp.float32)
            xk = xv * (gate * sel)
            pk = jnp.dot(xk, ew_ref[k], preferred_element_type=jnp.float32)
            acc = pk if acc is None else acc + pk
        out_ref[:, :] = acc

        for s in range(N_DEV - 1):
            c_send = lax.rem(p - s + 2 * N_DEV, N_DEV)
            rdma = pltpu.make_async_remote_copy(
                src_ref=out_ref.at[pl.ds(c_send * CHUNK, CHUNK), :],
                dst_ref=rs_buf.at[s],
                send_sem=rs_send.at[s],
                recv_sem=rs_recv.at[s],
                device_id=(right,),
                device_id_type=pl.DeviceIdType.MESH,
            )
            rdma.start()
            rdma.wait()
            c_recv = lax.rem(p - s - 1 + 2 * N_DEV, N_DEV)
            rows = pl.ds(c_recv * CHUNK, CHUNK)
            out_ref[rows, :] = out_ref[rows, :] + rs_buf[s]

        for d in range(1, N_DEV):
            tgt = lax.rem(p + d, N_DEV)
            pl.semaphore_signal(
                mid_sem, inc=1,
                device_id=(tgt,), device_id_type=pl.DeviceIdType.MESH,
            )
        pl.semaphore_wait(mid_sem, N_DEV - 1)

        for s in range(N_DEV - 1):
            c_send = lax.rem(p + 1 - s + 2 * N_DEV, N_DEV)
            rdma = pltpu.make_async_remote_copy(
                src_ref=out_ref.at[pl.ds(c_send * CHUNK, CHUNK), :],
                dst_ref=out_ref.at[pl.ds(c_send * CHUNK, CHUNK), :],
                send_sem=ag_send.at[s],
                recv_sem=ag_recv.at[s],
                device_id=(right,),
                device_id_type=pl.DeviceIdType.MESH,
            )
            rdma.start()
            rdma.wait()

        blk = 256
        for b in range(N_TOK // blk):
            rows = slice(b * blk, (b + 1) * blk)
            out_ref[rows, :] = out_ref[rows, :] + jnp.dot(
                xv[rows, :], sw_ref[:, :], preferred_element_type=jnp.float32
            )

    return pl.pallas_call(
        body,
        out_shape=jax.ShapeDtypeStruct((N_TOK, D_OUT), jnp.float32),
        in_specs=[pl.BlockSpec(memory_space=pltpu.VMEM)] * 5,
        out_specs=pl.BlockSpec(memory_space=pltpu.VMEM),
        scratch_shapes=[
            pltpu.VMEM((N_DEV - 1, CHUNK, D_OUT), jnp.float32),
            pltpu.SemaphoreType.DMA((N_DEV - 1,)),
            pltpu.SemaphoreType.DMA((N_DEV - 1,)),
            pltpu.SemaphoreType.DMA((N_DEV - 1,)),
            pltpu.SemaphoreType.DMA((N_DEV - 1,)),
            pltpu.SemaphoreType.REGULAR,
        ],
        compiler_params=pltpu.CompilerParams(collective_id=0),
    )(x, router_W, route_idx, expert_W, shared_W)


# device time: 123379 ns/iter; 1.3380x vs baseline; 1.3380x over previous
import jax
import jax.numpy as jnp
from jax import lax
from jax.experimental import pallas as pl
from jax.experimental.pallas import tpu as pltpu

N_DEV = 16
N_TOK = 1024
D_IN = 512
D_OUT = 1024
E_LOCAL = 4
BLK = 256

_MESH = pl.DeviceIdType.MESH


def kernel(x, router_W, route_idx, expert_W, shared_W):
    def body(x_ref, rw_ref, idx_ref, ew_ref, sw_ref, out_ref,
             prs_buf, za_buf, zb_buf, shared_buf,
             prs_send, prs_recv, z_send, z_recv, pag_send, pag_recv,
             plane_sem):
        p = lax.axis_index("i")
        g = p // 4
        q = lax.rem(p, 4)
        pb = g * 4
        right = pb + lax.rem(q + 1, 4)
        left = pb + lax.rem(q + 3, 4)
        pz1 = jnp.bitwise_xor(p, 4)
        pz2 = jnp.bitwise_xor(p, 8)
        z0 = lax.rem(g, 2)
        z1 = g // 2

        bsem = pltpu.get_barrier_semaphore()
        for nbr in (left, right, pz1, pz2):
            pl.semaphore_signal(bsem, inc=1, device_id=(nbr,),
                                device_id_type=_MESH)
        pl.semaphore_wait(bsem, 4)

        base = p * E_LOCAL

        def compute_block(row_off):
            xb = x_ref[pl.ds(row_off, BLK), :]
            idxb = idx_ref[pl.ds(row_off, BLK), :]
            sb = jnp.dot(xb, rw_ref[:, :], preferred_element_type=jnp.float32)
            mb = jnp.max(sb, axis=1, keepdims=True)
            gb = 1.0 / jnp.sum(jnp.exp(sb - mb), axis=1, keepdims=True)
            acc = None
            for k in range(E_LOCAL):
                sel = (idxb == base + k).astype(jnp.float32)
                xk = xb * (gb * sel)
                pk = jnp.dot(xk, ew_ref[k], preferred_element_type=jnp.float32)
                acc = pk if acc is None else acc + pk
            out_ref[pl.ds(row_off, BLK), :] = acc

        def shared_block(b):
            rows = slice(b * BLK, (b + 1) * BLK)
            shared_buf[rows, :] = jnp.dot(
                x_ref[rows, :], sw_ref[:, :],
                preferred_element_type=jnp.float32)

        def blk_off(s):
            return lax.rem(q - s + 8, 4) * BLK

        compute_block(blk_off(0))
        for s in range(3):
            rdma = pltpu.make_async_remote_copy(
                src_ref=out_ref.at[pl.ds(blk_off(s), BLK), :],
                dst_ref=prs_buf.at[s],
                send_sem=prs_send.at[s], recv_sem=prs_recv.at[s],
                device_id=(right,), device_id_type=_MESH,
            )
            rdma.start()
            compute_block(blk_off(s + 1))
            if s == 2:
                shared_block(0)
                shared_block(1)
            rdma.wait()
            ro = blk_off(s + 1)
            out_ref[pl.ds(ro, BLK), :] = out_ref[pl.ds(ro, BLK), :] + prs_buf[s]

        rb1 = lax.rem(q + 1, 4) * BLK

        o2 = rb1 + 128 * z0
        s2 = rb1 + 128 * (1 - z0)
        a = pltpu.make_async_remote_copy(
            src_ref=out_ref.at[pl.ds(s2, 128), :], dst_ref=za_buf,
            send_sem=z_send.at[0], recv_sem=z_recv.at[0],
            device_id=(pz1,), device_id_type=_MESH,
        )
        a.start()
        shared_block(2)
        shared_block(3)
        a.wait()
        out_ref[pl.ds(o2, 128), :] = out_ref[pl.ds(o2, 128), :] + za_buf[:, :]

        o3 = o2 + 64 * z1
        s3 = o2 + 64 * (1 - z1)
        b = pltpu.make_async_remote_copy(
            src_ref=out_ref.at[pl.ds(s3, 64), :], dst_ref=zb_buf,
            send_sem=z_send.at[1], recv_sem=z_recv.at[1],
            device_id=(pz2,), device_id_type=_MESH,
        )
        b.start()
        b.wait()
        out_ref[pl.ds(o3, 64), :] = out_ref[pl.ds(o3, 64), :] + zb_buf[:, :]

        bp = pltpu.make_async_remote_copy(
            src_ref=out_ref.at[pl.ds(o3, 64), :],
            dst_ref=out_ref.at[pl.ds(o3, 64), :],
            send_sem=z_send.at[2], recv_sem=z_recv.at[2],
            device_id=(pz2,), device_id_type=_MESH,
        )
        bp.start()
        bp.wait()
        ap = pltpu.make_async_remote_copy(
            src_ref=out_ref.at[pl.ds(o2, 128), :],
            dst_ref=out_ref.at[pl.ds(o2, 128), :],
            send_sem=z_send.at[3], recv_sem=z_recv.at[3],
            device_id=(pz1,), device_id_type=_MESH,
        )
        ap.start()
        ap.wait()

        for d in (1, 2, 3):
            tgt = pb + lax.rem(q + d, 4)
            pl.semaphore_signal(plane_sem, inc=1, device_id=(tgt,),
                                device_id_type=_MESH)
        pl.semaphore_wait(plane_sem, 3)

        for s in range(3):
            off = lax.rem(q + 1 - s + 8, 4) * BLK
            rdma = pltpu.make_async_remote_copy(
                src_ref=out_ref.at[pl.ds(off, BLK), :],
                dst_ref=out_ref.at[pl.ds(off, BLK), :],
                send_sem=pag_send.at[s], recv_sem=pag_recv.at[s],
                device_id=(right,), device_id_type=_MESH,
            )
            rdma.start()
            rdma.wait()

        for bb in range(4):
            rows = slice(bb * BLK, (bb + 1) * BLK)
            out_ref[rows, :] = out_ref[rows, :] + shared_buf[rows, :]

    return pl.pallas_call(
        body,
        out_shape=jax.ShapeDtypeStruct((N_TOK, D_OUT), jnp.float32),
        in_specs=[pl.BlockSpec(memory_space=pltpu.VMEM)] * 5,
        out_specs=pl.BlockSpec(memory_space=pltpu.VMEM),
        scratch_shapes=[
            pltpu.VMEM((3, BLK, D_OUT), jnp.float32),
            pltpu.VMEM((128, D_OUT), jnp.float32),
            pltpu.VMEM((64, D_OUT), jnp.float32),
            pltpu.VMEM((N_TOK, D_OUT), jnp.float32),
            pltpu.SemaphoreType.DMA((3,)),
            pltpu.SemaphoreType.DMA((3,)),
            pltpu.SemaphoreType.DMA((4,)),
            pltpu.SemaphoreType.DMA((4,)),
            pltpu.SemaphoreType.DMA((3,)),
            pltpu.SemaphoreType.DMA((3,)),
            pltpu.SemaphoreType.REGULAR,
        ],
        compiler_params=pltpu.CompilerParams(collective_id=0),
    )(x, router_W, route_idx, expert_W, shared_W)


# device time: 58388 ns/iter; 2.8274x vs baseline; 2.1131x over previous
import jax
import jax.numpy as jnp
from jax import lax
from jax.experimental import pallas as pl
from jax.experimental.pallas import tpu as pltpu

N_DEV = 16
N_TOK = 1024
D_IN = 512
D_OUT = 1024
E_LOCAL = 4
BLK = 256
CHUNK = 64

_MESH = pl.DeviceIdType.MESH


def kernel(x, router_W, route_idx, expert_W, shared_W):
    def body(x_ref, rw_ref, idx_ref, ew_ref, sw_ref, out_ref,
             red_buf, abuf, bbuf, shared_buf, xbf, ewb, swb,
             a_send, a_recv, b_send, b_recv, c_send, c_recv,
             d_send, d_recv):
        p = lax.axis_index("i")
        g = p // 4
        q = lax.rem(p, 4)
        pb = g * 4

        bsem = pltpu.get_barrier_semaphore()
        for j in (1, 2, 3):
            pl.semaphore_signal(bsem, inc=1,
                                device_id=(pb + lax.rem(q + j, 4),),
                                device_id_type=_MESH)
            pl.semaphore_signal(bsem, inc=1,
                                device_id=(lax.rem(p + 4 * j, N_DEV),),
                                device_id_type=_MESH)
        xbf[:, :] = x_ref[:, :].astype(jnp.bfloat16)
        ewb[:, :, :] = ew_ref[:, :, :].astype(jnp.bfloat16)
        swb[:, :] = sw_ref[:, :].astype(jnp.bfloat16)
        pl.semaphore_wait(bsem, 6)

        base = p * E_LOCAL

        def compute_block(b):
            rows = slice(b * BLK, (b + 1) * BLK)
            xb32 = x_ref[rows, :]
            idxb = idx_ref[rows, :]
            sb = jnp.dot(xb32, rw_ref[:, :],
                         preferred_element_type=jnp.float32)
            mb = jnp.max(sb, axis=1, keepdims=True)
            gb = 1.0 / jnp.sum(jnp.exp(sb - mb), axis=1, keepdims=True)
            acc = None
            for k in range(E_LOCAL):
                sel = (idxb == base + k).astype(jnp.float32)
                xk = (xb32 * (gb * sel)).astype(jnp.bfloat16)
                pk = jnp.dot(xk, ewb[k], preferred_element_type=jnp.float32)
                acc = pk if acc is None else acc + pk
            red_buf[4 * b:4 * (b + 1)] = acc.astype(jnp.bfloat16).reshape(
                4, CHUNK, D_OUT)

        def shared_block(b):
            rows = slice(b * BLK, (b + 1) * BLK)
            shared_buf[rows, :] = jnp.dot(
                xbf[rows, :], swb[:, :], preferred_element_type=jnp.float32)

        a_rdma = []
        for b in range(4):
            compute_block(b)
            for j in (1, 2, 3):
                qr = lax.rem(q + j, 4)
                slot = (j - 1) * 4 + b
                rdma = pltpu.make_async_remote_copy(
                    src_ref=red_buf.at[4 * b + qr],
                    dst_ref=abuf.at[slot],
                    send_sem=a_send.at[slot], recv_sem=a_recv.at[slot],
                    device_id=(pb + qr,), device_id_type=_MESH,
                )
                rdma.start()
                a_rdma.append(rdma)
        shared_block(0)
        shared_block(1)
        for rdma in a_rdma:
            rdma.wait()
        for b in range(4):
            cq = 4 * b + q
            s = red_buf[cq].astype(jnp.float32)
            for j in (1, 2, 3):
                s = s + abuf[(j - 1) * 4 + b].astype(jnp.float32)
            red_buf[cq] = s.astype(jnp.bfloat16)

        b_rdma = []
        for t in (1, 2, 3):
            gm = lax.rem(g + t, 4)
            m = lax.rem(p + 4 * t, N_DEV)
            rdma = pltpu.make_async_remote_copy(
                src_ref=red_buf.at[4 * gm + q],
                dst_ref=bbuf.at[t - 1],
                send_sem=b_send.at[t - 1], recv_sem=b_recv.at[t - 1],
                device_id=(m,), device_id_type=_MESH,
            )
            rdma.start()
            b_rdma.append(rdma)
        shared_block(2)
        shared_block(3)
        for rdma in b_rdma:
            rdma.wait()
        cp = 4 * g + q
        s = red_buf[cp].astype(jnp.float32)
        for t in (1, 2, 3):
            s = s + bbuf[t - 1].astype(jnp.float32)
        red_buf[cp] = s.astype(jnp.bfloat16)

        c_rdma = []
        for t in (1, 2, 3):
            m = lax.rem(p + 4 * t, N_DEV)
            rdma = pltpu.make_async_remote_copy(
                src_ref=red_buf.at[cp], dst_ref=red_buf.at[cp],
                send_sem=c_send.at[t - 1], recv_sem=c_recv.at[t - 1],
                device_id=(m,), device_id_type=_MESH,
            )
            rdma.start()
            c_rdma.append(rdma)
        for rdma in c_rdma:
            rdma.wait()

        d_rdma = []
        for t in range(4):
            ct = 4 * lax.rem(g + t, 4) + q
            for j in (1, 2, 3):
                slot = t * 3 + (j - 1)
                rdma = pltpu.make_async_remote_copy(
                    src_ref=red_buf.at[ct], dst_ref=red_buf.at[ct],
                    send_sem=d_send.at[slot], recv_sem=d_recv.at[slot],
                    device_id=(pb + lax.rem(q + j, 4),),
                    device_id_type=_MESH,
                )
                rdma.start()
                d_rdma.append(rdma)
        for rdma in d_rdma:
            rdma.wait()

        for b in range(4):
            rows = slice(b * BLK, (b + 1) * BLK)
            out_ref[rows, :] = (
                red_buf[4 * b:4 * (b + 1)].reshape(BLK, D_OUT).astype(
                    jnp.float32) + shared_buf[rows, :])

    return pl.pallas_call(
        body,
        out_shape=jax.ShapeDtypeStruct((N_TOK, D_OUT), jnp.float32),
        in_specs=[pl.BlockSpec(memory_space=pltpu.VMEM)] * 5,
        out_specs=pl.BlockSpec(memory_space=pltpu.VMEM),
        scratch_shapes=[
            pltpu.VMEM((N_DEV, CHUNK, D_OUT), jnp.bfloat16),
            pltpu.VMEM((12, CHUNK, D_OUT), jnp.bfloat16),
            pltpu.VMEM((3, CHUNK, D_OUT), jnp.bfloat16),
            pltpu.VMEM((N_TOK, D_OUT), jnp.float32),
            pltpu.VMEM((N_TOK, D_IN), jnp.bfloat16),
            pltpu.VMEM((E_LOCAL, D_IN, D_OUT), jnp.bfloat16),
            pltpu.VMEM((D_IN, D_OUT), jnp.bfloat16),
            pltpu.SemaphoreType.DMA((12,)),
            pltpu.SemaphoreType.DMA((12,)),
            pltpu.SemaphoreType.DMA((3,)),
            pltpu.SemaphoreType.DMA((3,)),
            pltpu.SemaphoreType.DMA((3,)),
            pltpu.SemaphoreType.DMA((3,)),
            pltpu.SemaphoreType.DMA((12,)),
            pltpu.SemaphoreType.DMA((12,)),
        ],
        compiler_params=pltpu.CompilerParams(collective_id=0),
    )(x, router_W, route_idx, expert_W, shared_W)


# device time: 53756 ns/iter; 3.0710x vs baseline; 1.0862x over previous
import jax
import jax.numpy as jnp
from jax import lax
from jax.experimental import pallas as pl
from jax.experimental.pallas import tpu as pltpu

N_DEV = 16
N_TOK = 1024
D_IN = 512
D_OUT = 1024
E_LOCAL = 4
BLK = 256
CHUNK = 64

_MESH = pl.DeviceIdType.MESH


def kernel(x, router_W, route_idx, expert_W, shared_W):
    def body(x_ref, rw_ref, idx_ref, ew_ref, sw_ref, out_ref,
             red_buf, abuf, bbuf, shared_buf, xbf, ewb, swb,
             a_send, a_recv, b_send, b_recv, c_send, c_recv,
             d_send, d_recv):
        p = lax.axis_index("i")
        g = p // 4
        q = lax.rem(p, 4)
        pb = g * 4

        bsem = pltpu.get_barrier_semaphore()
        for j in (1, 2, 3):
            pl.semaphore_signal(bsem, inc=1,
                                device_id=(pb + lax.rem(q + j, 4),),
                                device_id_type=_MESH)
            pl.semaphore_signal(bsem, inc=1,
                                device_id=(lax.rem(p + 4 * j, N_DEV),),
                                device_id_type=_MESH)
        xbf[:, :] = x_ref[:, :].astype(jnp.bfloat16)
        ewb[:, :, :] = ew_ref[:, :, :].astype(jnp.bfloat16)
        swb[:, :] = sw_ref[:, :].astype(jnp.bfloat16)
        pl.semaphore_wait(bsem, 6)

        base = p * E_LOCAL

        def compute_block(b):
            rows = slice(b * BLK, (b + 1) * BLK)
            xb32 = x_ref[rows, :]
            idxb = idx_ref[rows, :]
            sb = jnp.dot(xb32, rw_ref[:, :],
                         preferred_element_type=jnp.float32)
            mb = jnp.max(sb, axis=1, keepdims=True)
            gb = 1.0 / jnp.sum(jnp.exp(sb - mb), axis=1, keepdims=True)
            acc = None
            for k in range(E_LOCAL):
                sel = (idxb == base + k).astype(jnp.float32)
                xk = (xb32 * (gb * sel)).astype(jnp.bfloat16)
                pk = jnp.dot(xk, ewb[k], preferred_element_type=jnp.float32)
                acc = pk if acc is None else acc + pk
            red_buf[4 * b:4 * (b + 1)] = acc.astype(jnp.bfloat16).reshape(
                4, CHUNK, D_OUT)

        def shared_block(b):
            rows = slice(b * BLK, (b + 1) * BLK)
            shared_buf[rows, :] = jnp.dot(
                xbf[rows, :], swb[:, :], preferred_element_type=jnp.float32)

        a_rdma = []
        for b in range(4):
            compute_block(b)
            for j in (1, 2, 3):
                qr = lax.rem(q + j, 4)
                slot = (j - 1) * 4 + b
                rdma = pltpu.make_async_remote_copy(
                    src_ref=red_buf.at[4 * b + qr],
                    dst_ref=abuf.at[slot],
                    send_sem=a_send.at[slot], recv_sem=a_recv.at[slot],
                    device_id=(pb + qr,), device_id_type=_MESH,
                )
                rdma.start()
                a_rdma.append(rdma)
        shared_block(0)
        shared_block(1)
        for b in range(4):
            for j in (1, 2, 3):
                a_rdma[b * 3 + (j - 1)].wait()
            cq = 4 * b + q
            s = red_buf[cq].astype(jnp.float32)
            for j in (1, 2, 3):
                s = s + abuf[(j - 1) * 4 + b].astype(jnp.float32)
            red_buf[cq] = s.astype(jnp.bfloat16)

        b_rdma = []
        for t in (1, 2, 3):
            gm = lax.rem(g + t, 4)
            m = lax.rem(p + 4 * t, N_DEV)
            rdma = pltpu.make_async_remote_copy(
                src_ref=red_buf.at[4 * gm + q],
                dst_ref=bbuf.at[t - 1],
                send_sem=b_send.at[t - 1], recv_sem=b_recv.at[t - 1],
                device_id=(m,), device_id_type=_MESH,
            )
            rdma.start()
            b_rdma.append(rdma)
        shared_block(2)
        shared_block(3)
        for rdma in b_rdma:
            rdma.wait()
        cp = 4 * g + q
        s = red_buf[cp].astype(jnp.float32)
        for t in (1, 2, 3):
            s = s + bbuf[t - 1].astype(jnp.float32)
        red_buf[cp] = s.astype(jnp.bfloat16)

        def issue_d(t):
            ct = 4 * lax.rem(g + t, 4) + q
            group = []
            for j in (1, 2, 3):
                slot = t * 3 + (j - 1)
                rdma = pltpu.make_async_remote_copy(
                    src_ref=red_buf.at[ct], dst_ref=red_buf.at[ct],
                    send_sem=d_send.at[slot], recv_sem=d_recv.at[slot],
                    device_id=(pb + lax.rem(q + j, 4),),
                    device_id_type=_MESH,
                )
                rdma.start()
                group.append(rdma)
            return group

        d_rdma = {0: issue_d(0)}
        c_rdma = []
        for t in (1, 2, 3):
            m = lax.rem(p + 4 * t, N_DEV)
            rdma = pltpu.make_async_remote_copy(
                src_ref=red_buf.at[cp], dst_ref=red_buf.at[cp],
                send_sem=c_send.at[t - 1], recv_sem=c_recv.at[t - 1],
                device_id=(m,), device_id_type=_MESH,
            )
            rdma.start()
            c_rdma.append(rdma)
        for t in (1, 2, 3):
            c_rdma[t - 1].wait()
            d_rdma[4 - t] = issue_d(4 - t)

        for t in range(4):
            for rdma in d_rdma[t]:
                rdma.wait()
            bt = lax.rem(g + t, 4)
            val = red_buf[pl.ds(4 * bt, 4), :, :]
            rows = pl.ds(bt * BLK, BLK)
            out_ref[rows, :] = (
                val.reshape(BLK, D_OUT).astype(jnp.float32)
                + shared_buf[rows, :])

    return pl.pallas_call(
        body,
        out_shape=jax.ShapeDtypeStruct((N_TOK, D_OUT), jnp.float32),
        in_specs=[pl.BlockSpec(memory_space=pltpu.VMEM)] * 5,
        out_specs=pl.BlockSpec(memory_space=pltpu.VMEM),
        scratch_shapes=[
            pltpu.VMEM((N_DEV, CHUNK, D_OUT), jnp.bfloat16),
            pltpu.VMEM((12, CHUNK, D_OUT), jnp.bfloat16),
            pltpu.VMEM((3, CHUNK, D_OUT), jnp.bfloat16),
            pltpu.VMEM((N_TOK, D_OUT), jnp.float32),
            pltpu.VMEM((N_TOK, D_IN), jnp.bfloat16),
            pltpu.VMEM((E_LOCAL, D_IN, D_OUT), jnp.bfloat16),
            pltpu.VMEM((D_IN, D_OUT), jnp.bfloat16),
            pltpu.SemaphoreType.DMA((12,)),
            pltpu.SemaphoreType.DMA((12,)),
            pltpu.SemaphoreType.DMA((3,)),
            pltpu.SemaphoreType.DMA((3,)),
            pltpu.SemaphoreType.DMA((3,)),
            pltpu.SemaphoreType.DMA((3,)),
            pltpu.SemaphoreType.DMA((12,)),
            pltpu.SemaphoreType.DMA((12,)),
        ],
        compiler_params=pltpu.CompilerParams(collective_id=0),
    )(x, router_W, route_idx, expert_W, shared_W)
